# baseline (speedup 1.0000x reference)
import functools

import jax
import jax.numpy as jnp
from jax import lax
from jax.experimental import pallas as pl
from jax.experimental.pallas import tpu as pltpu
from jax.experimental.pallas import tpu_sc as plsc

S = 2048
D = 1024
F = 4096
NC, NS, K = 4, 4, 2
E = NC * NS
T = 512
R = S * K + E * T
NT = R // T
FB = 2048
NF = F // FB

NW = 32
XCH = 32
TOK_PER_W = S // NW
CCH = 16


@functools.cache
def _mesh():
    return plsc.VectorSubcoreMesh(core_axis_name="c", subcore_axis_name="s")


def _routing(cat_indices, sub_indices, weights):
    eid = (cat_indices * NS + sub_indices).reshape(S * K).astype(jnp.int32)
    w = weights.reshape(S * K)
    onehot = (eid[:, None] == jnp.arange(E, dtype=jnp.int32)[None, :]).astype(jnp.int16)
    ranks = jnp.cumsum(onehot, axis=0)
    counts = ranks[-1].astype(jnp.int32)
    rank = jnp.take_along_axis(ranks, eid[:, None], axis=1)[:, 0].astype(jnp.int32) - 1
    padded = ((counts + T - 1) // T) * T
    cum = jnp.cumsum(padded)
    offs = cum - padded
    total = cum[-1].astype(jnp.int32)
    dest = (offs[eid] + rank).astype(jnp.int32)
    tile_start = jnp.arange(NT, dtype=jnp.int32) * T
    tile_e = jnp.searchsorted(
        cum, jnp.minimum(tile_start, total - T), side="right").astype(jnp.int32)
    tile_active = (tile_start < total).astype(jnp.int32)
    return tile_e, tile_active, dest, total


def _sc_scatter_x(h2, dest0_3, dest1_3):
    NCH = TOK_PER_W // XCH

    @functools.partial(
        pl.kernel, mesh=_mesh(),
        out_type=jax.ShapeDtypeStruct((R, D), jnp.float32),
        scratch_types=[pltpu.VMEM((XCH,), jnp.int32) for _ in range(4)]
          + [pltpu.VMEM((XCH, D), jnp.float32) for _ in range(2)]
          + [pltpu.SemaphoreType.DMA for _ in range(4)],
    )
    def k(h_hbm, d0_hbm, d1_hbm, x_hbm, *sc):
        i0s, i1s = sc[0:2], sc[2:4]
        bufs = sc[4:6]
        s0s, s1s = sc[6:8], sc[8:10]
        wid = lax.axis_index("s") * 2 + lax.axis_index("c")
        base = wid * TOK_PER_W

        def start(c):
            p = c % 2
            pltpu.sync_copy(h_hbm.at[pl.ds(base + c * XCH, XCH)], bufs[p])
            pltpu.sync_copy(d0_hbm.at[wid, c], i0s[p])
            pltpu.sync_copy(d1_hbm.at[wid, c], i1s[p])
            da = pltpu.async_copy(bufs[p], x_hbm.at[i0s[p]], s0s[p])
            db = pltpu.async_copy(bufs[p], x_hbm.at[i1s[p]], s1s[p])
            return da, db

        cur = start(0)
        for c in range(NCH):
            nxt = start(c + 1) if c + 1 < NCH else None
            cur[0].wait()
            cur[1].wait()
            cur = nxt

    return k(h2, dest0_3, dest1_3)


def _tc_expert_mlp(x, w1, b1, w2, b2, tile_e, tile_active, nt_dyn):
    def body(te_ref, ta_ref, x_ref, w1_ref, b1_ref, w2_ref, b2_ref, y_ref):
        t = pl.program_id(0)
        f = pl.program_id(1)

        @pl.when(f == 0)
        def _():
            y_ref[...] = jnp.broadcast_to(b2_ref[0, 0][None, :], (T, D))

        @pl.when(ta_ref[t] == 1)
        def _():
            h = jnp.dot(x_ref[...].astype(jnp.bfloat16),
                        w1_ref[0].astype(jnp.bfloat16),
                        preferred_element_type=jnp.float32)
            h = jnp.maximum(h + b1_ref[0, 0][None, :], 0.0)
            y_ref[...] += jnp.dot(h.astype(jnp.bfloat16),
                                  w2_ref[0].astype(jnp.bfloat16),
                                  preferred_element_type=jnp.float32)

    grid_spec = pltpu.PrefetchScalarGridSpec(
        num_scalar_prefetch=2,
        grid=(nt_dyn, NF),
        in_specs=[
            pl.BlockSpec((T, D), lambda t, f, te, ta: (t, 0)),
            pl.BlockSpec((1, D, FB), lambda t, f, te, ta: (te[t], 0, f)),
            pl.BlockSpec((1, 1, FB), lambda t, f, te, ta: (te[t] * NF + f, 0, 0)),
            pl.BlockSpec((1, FB, D), lambda t, f, te, ta: (te[t], f, 0)),
            pl.BlockSpec((1, 1, D), lambda t, f, te, ta: (te[t], 0, 0)),
        ],
        out_specs=pl.BlockSpec((T, D), lambda t, f, te, ta: (t, 0)),
    )
    return pl.pallas_call(
        body,
        grid_spec=grid_spec,
        out_shape=jax.ShapeDtypeStruct((R, D), jnp.float32),
        compiler_params=pltpu.CompilerParams(
            dimension_semantics=("arbitrary", "arbitrary")),
    )(tile_e, tile_active, x, w1, b1.reshape(E * NF, 1, FB), w2,
      b2.reshape(E, 1, D))


def _sc_combine(y, pos0_3, pos1_3, w0b, w1b):
    NCC = TOK_PER_W // CCH

    @functools.partial(
        pl.kernel, mesh=_mesh(),
        out_type=jax.ShapeDtypeStruct((S, D), jnp.float32),
        scratch_types=[pltpu.VMEM((CCH,), jnp.int32) for _ in range(6)]
          + [pltpu.VMEM((CCH, D), jnp.float32) for _ in range(6)]
          + [pltpu.VMEM((CCH, 16), jnp.float32) for _ in range(6)]
          + [pltpu.SemaphoreType.DMA for _ in range(6)],
    )
    def k(y_hbm, p0_hbm, p1_hbm, w0_hbm, w1_hbm, out_hbm, *sc):
        i0buf, i1buf = sc[0:3], sc[3:6]
        abuf, bbuf = sc[6:9], sc[9:12]
        w0buf, w1buf = sc[12:15], sc[15:18]
        asem, bsem = sc[18:21], sc[21:24]
        wid = lax.axis_index("s") * 2 + lax.axis_index("c")
        base = wid * TOK_PER_W

        def start(c):
            p = c % 3
            pltpu.sync_copy(p0_hbm.at[wid, c], i0buf[p])
            pltpu.sync_copy(p1_hbm.at[wid, c], i1buf[p])
            pltpu.sync_copy(w0_hbm.at[wid, c], w0buf[p])
            pltpu.sync_copy(w1_hbm.at[wid, c], w1buf[p])
            da = pltpu.async_copy(y_hbm.at[i0buf[p]], abuf[p], asem[p])
            db = pltpu.async_copy(y_hbm.at[i1buf[p]], bbuf[p], bsem[p])
            return da, db

        descs = {0: start(0), 1: start(1)}
        for c in range(NCC):
            if c + 2 < NCC:
                descs[c + 2] = start(c + 2)
            descs[c][0].wait()
            descs[c][1].wait()
            p = c % 3
            av, bv, w0v, w1v = abuf[p], bbuf[p], w0buf[p], w1buf[p]

            def row(i, carry, av=av, bv=bv, w0v=w0v, w1v=w1v):
                w0r = w0v[i, :]
                w1r = w1v[i, :]
                for j in range(D // 16):
                    sl = pl.ds(j * 16, 16)
                    av[i, sl] = av[i, sl] * w0r + bv[i, sl] * w1r
                return carry

            lax.fori_loop(0, CCH, row, 0)
            pltpu.sync_copy(av, out_hbm.at[pl.ds(base + c * CCH, CCH)])

    return k(y, pos0_3, pos1_3, w0b, w1b)


def kernel(hidden, cat_indices, sub_indices, weights, W1, b1, W2, b2):
    h2 = hidden.reshape(S, D)
    tile_e, tile_active, dest, total = _routing(cat_indices, sub_indices, weights)

    pos = dest.reshape(S, K)
    dest0_3 = pos[:, 0].reshape(NW, TOK_PER_W // XCH, XCH)
    dest1_3 = pos[:, 1].reshape(NW, TOK_PER_W // XCH, XCH)
    x = _sc_scatter_x(h2, dest0_3, dest1_3)

    y = _tc_expert_mlp(x, W1, b1, W2, b2, tile_e, tile_active, total // T)

    pos0_3 = pos[:, 0].reshape(NW, TOK_PER_W // CCH, CCH)
    pos1_3 = pos[:, 1].reshape(NW, TOK_PER_W // CCH, CCH)
    wflat = weights.reshape(S, K)
    w0b = jnp.broadcast_to(wflat[:, 0][:, None], (S, 16)).reshape(
        NW, TOK_PER_W // CCH, CCH, 16)
    w1b = jnp.broadcast_to(wflat[:, 1][:, None], (S, 16)).reshape(
        NW, TOK_PER_W // CCH, CCH, 16)
    out = _sc_combine(y, pos0_3, pos1_3, w0b, w1b)
    return out.reshape(hidden.shape)


# speedup vs baseline: 1.0232x; 1.0232x over previous
import functools

import jax
import jax.numpy as jnp
from jax import lax
from jax.experimental import pallas as pl
from jax.experimental.pallas import tpu as pltpu
from jax.experimental.pallas import tpu_sc as plsc

S = 2048
D = 1024
F = 4096
NC, NS, K = 4, 4, 2
E = NC * NS
T = 512
R = S * K + E * T
NT = R // T
FB = 2048
NF = F // FB

NW = 32
TOK_PER_W = S // NW
CCH = 16


@functools.cache
def _mesh():
    return plsc.VectorSubcoreMesh(core_axis_name="c", subcore_axis_name="s")


def _routing(cat_indices, sub_indices, weights):
    eid = (cat_indices * NS + sub_indices).reshape(S * K).astype(jnp.int32)
    w = weights.reshape(S * K)
    onehot = (eid[:, None] == jnp.arange(E, dtype=jnp.int32)[None, :]).astype(jnp.int16)
    ranks = jnp.cumsum(onehot, axis=0)
    counts = ranks[-1].astype(jnp.int32)
    rank = jnp.take_along_axis(ranks, eid[:, None], axis=1)[:, 0].astype(jnp.int32) - 1
    padded = ((counts + T - 1) // T) * T
    cum = jnp.cumsum(padded)
    offs = cum - padded
    total = cum[-1].astype(jnp.int32)
    dest = (offs[eid] + rank).astype(jnp.int32)
    tile_start = jnp.arange(NT, dtype=jnp.int32) * T
    tile_e = jnp.searchsorted(
        cum, jnp.minimum(tile_start, total - T), side="right").astype(jnp.int32)
    tile_active = (tile_start < total).astype(jnp.int32)
    return tile_e, tile_active, dest, total


def _sc_scatter_x(h2, dest0_2, dest1_2):
    @functools.partial(
        pl.kernel, mesh=_mesh(),
        out_type=jax.ShapeDtypeStruct((R, D), jnp.float32),
        scratch_types=[
            pltpu.VMEM((TOK_PER_W,), jnp.int32),
            pltpu.VMEM((TOK_PER_W,), jnp.int32),
            pltpu.VMEM((TOK_PER_W, D), jnp.float32),
            pltpu.SemaphoreType.DMA,
            pltpu.SemaphoreType.DMA,
        ],
    )
    def k(h_hbm, d0_hbm, d1_hbm, x_hbm, i0, i1, buf, s0, s1):
        wid = lax.axis_index("s") * 2 + lax.axis_index("c")
        base = wid * TOK_PER_W
        pltpu.sync_copy(d0_hbm.at[wid], i0)
        pltpu.sync_copy(d1_hbm.at[wid], i1)
        pltpu.sync_copy(h_hbm.at[pl.ds(base, TOK_PER_W)], buf)
        da = pltpu.async_copy(buf, x_hbm.at[i0], s0)
        db = pltpu.async_copy(buf, x_hbm.at[i1], s1)
        da.wait()
        db.wait()

    return k(h2, dest0_2, dest1_2)


def _tc_expert_mlp(x, w1, b1, w2, b2, tile_e, tile_active, nt_dyn):
    def body(te_ref, ta_ref, x_ref, w1_ref, b1_ref, w2_ref, b2_ref, y_ref):
        t = pl.program_id(0)
        f = pl.program_id(1)

        @pl.when(f == 0)
        def _():
            y_ref[...] = jnp.broadcast_to(b2_ref[0, 0][None, :], (T, D))

        @pl.when(ta_ref[t] == 1)
        def _():
            h = jnp.dot(x_ref[...].astype(jnp.bfloat16),
                        w1_ref[0].astype(jnp.bfloat16),
                        preferred_element_type=jnp.float32)
            h = jnp.maximum(h + b1_ref[0, 0][None, :], 0.0)
            y_ref[...] += jnp.dot(h.astype(jnp.bfloat16),
                                  w2_ref[0].astype(jnp.bfloat16),
                                  preferred_element_type=jnp.float32)

    grid_spec = pltpu.PrefetchScalarGridSpec(
        num_scalar_prefetch=2,
        grid=(nt_dyn, NF),
        in_specs=[
            pl.BlockSpec((T, D), lambda t, f, te, ta: (t, 0)),
            pl.BlockSpec((1, D, FB), lambda t, f, te, ta: (te[t], 0, f)),
            pl.BlockSpec((1, 1, FB), lambda t, f, te, ta: (te[t] * NF + f, 0, 0)),
            pl.BlockSpec((1, FB, D), lambda t, f, te, ta: (te[t], f, 0)),
            pl.BlockSpec((1, 1, D), lambda t, f, te, ta: (te[t], 0, 0)),
        ],
        out_specs=pl.BlockSpec((T, D), lambda t, f, te, ta: (t, 0)),
    )
    return pl.pallas_call(
        body,
        grid_spec=grid_spec,
        out_shape=jax.ShapeDtypeStruct((R, D), jnp.float32),
        compiler_params=pltpu.CompilerParams(
            dimension_semantics=("arbitrary", "arbitrary")),
    )(tile_e, tile_active, x, w1, b1.reshape(E * NF, 1, FB), w2,
      b2.reshape(E, 1, D))


def _sc_combine(y, pos0_2, pos1_2, w0b, w1b):
    NCC = TOK_PER_W // CCH

    @functools.partial(
        pl.kernel, mesh=_mesh(),
        out_type=jax.ShapeDtypeStruct((S, D), jnp.float32),
        scratch_types=[
            pltpu.VMEM((TOK_PER_W,), jnp.int32),
            pltpu.VMEM((TOK_PER_W,), jnp.int32),
            pltpu.VMEM((TOK_PER_W, 16), jnp.float32),
            pltpu.VMEM((TOK_PER_W, 16), jnp.float32),
        ] + [pltpu.VMEM((CCH, D), jnp.float32) for _ in range(6)]
          + [pltpu.SemaphoreType.DMA for _ in range(6)],
    )
    def k(y_hbm, p0_hbm, p1_hbm, w0_hbm, w1_hbm, out_hbm, p0v, p1v, w0v, w1v, *sc):
        abuf, bbuf = sc[0:3], sc[3:6]
        asem, bsem = sc[6:9], sc[9:12]
        wid = lax.axis_index("s") * 2 + lax.axis_index("c")
        base = wid * TOK_PER_W
        pltpu.sync_copy(p0_hbm.at[wid], p0v)
        pltpu.sync_copy(p1_hbm.at[wid], p1v)
        pltpu.sync_copy(w0_hbm.at[wid], w0v)
        pltpu.sync_copy(w1_hbm.at[wid], w1v)

        def start(c):
            p = c % 3
            sl = pl.ds(c * CCH, CCH)
            da = pltpu.async_copy(y_hbm.at[p0v.at[sl]], abuf[p], asem[p])
            db = pltpu.async_copy(y_hbm.at[p1v.at[sl]], bbuf[p], bsem[p])
            return da, db

        descs = {0: start(0), 1: start(1)}
        for c in range(NCC):
            if c + 2 < NCC:
                descs[c + 2] = start(c + 2)
            descs[c][0].wait()
            descs[c][1].wait()
            p = c % 3
            av, bv = abuf[p], bbuf[p]

            def row(i, carry, av=av, bv=bv, c=c):
                w0r = w0v[c * CCH + i, :]
                w1r = w1v[c * CCH + i, :]
                for j in range(D // 16):
                    sl = pl.ds(j * 16, 16)
                    av[i, sl] = av[i, sl] * w0r + bv[i, sl] * w1r
                return carry

            lax.fori_loop(0, CCH, row, 0)
            pltpu.sync_copy(av, out_hbm.at[pl.ds(base + c * CCH, CCH)])

    return k(y, pos0_2, pos1_2, w0b, w1b)


def kernel(hidden, cat_indices, sub_indices, weights, W1, b1, W2, b2):
    h2 = hidden.reshape(S, D)
    tile_e, tile_active, dest, total = _routing(cat_indices, sub_indices, weights)

    pos = dest.reshape(S, K)
    dest0_2 = pos[:, 0].reshape(NW, TOK_PER_W)
    dest1_2 = pos[:, 1].reshape(NW, TOK_PER_W)
    x = _sc_scatter_x(h2, dest0_2, dest1_2)

    y = _tc_expert_mlp(x, W1, b1, W2, b2, tile_e, tile_active, total // T)

    wflat = weights.reshape(S, K)
    w0b = jnp.broadcast_to(wflat[:, 0][:, None], (S, 16)).reshape(
        NW, TOK_PER_W, 16)
    w1b = jnp.broadcast_to(wflat[:, 1][:, None], (S, 16)).reshape(
        NW, TOK_PER_W, 16)
    out = _sc_combine(y, dest0_2, dest1_2, w0b, w1b)
    return out.reshape(hidden.shape)


# speedup vs baseline: 1.0341x; 1.0106x over previous
import functools

import jax
import jax.numpy as jnp
from jax import lax
from jax.experimental import pallas as pl
from jax.experimental.pallas import tpu as pltpu
from jax.experimental.pallas import tpu_sc as plsc

S = 2048
D = 1024
F = 4096
NC, NS, K = 4, 4, 2
E = NC * NS
T = 512
R = S * K + E * T
NT = R // T
FB = 2048
NF = F // FB

NW = 32
TOK_PER_W = S // NW
CCH = 16


@functools.cache
def _mesh():
    return plsc.VectorSubcoreMesh(core_axis_name="c", subcore_axis_name="s")


def _routing(cat_indices, sub_indices, weights):
    eid = (cat_indices * NS + sub_indices).reshape(S * K).astype(jnp.int32)
    w = weights.reshape(S * K)
    onehot = (eid[:, None] == jnp.arange(E, dtype=jnp.int32)[None, :]).astype(jnp.int16)
    ranks = jnp.cumsum(onehot, axis=0)
    counts = ranks[-1].astype(jnp.int32)
    rank = jnp.take_along_axis(ranks, eid[:, None], axis=1)[:, 0].astype(jnp.int32) - 1
    padded = ((counts + T - 1) // T) * T
    cum = jnp.cumsum(padded)
    offs = cum - padded
    total = cum[-1].astype(jnp.int32)
    dest = (offs[eid] + rank).astype(jnp.int32)
    tile_start = jnp.arange(NT, dtype=jnp.int32) * T
    tstart = jnp.minimum(tile_start, total - T)
    tile_e = jnp.sum((cum[None, :] <= tstart[:, None]).astype(jnp.int32), axis=1)
    tile_active = (tile_start < total).astype(jnp.int32)
    return tile_e, tile_active, dest, total


def _sc_scatter_x(h2, dest0_2, dest1_2):
    @functools.partial(
        pl.kernel, mesh=_mesh(),
        out_type=jax.ShapeDtypeStruct((R, D), jnp.float32),
        scratch_types=[
            pltpu.VMEM((TOK_PER_W,), jnp.int32),
            pltpu.VMEM((TOK_PER_W,), jnp.int32),
            pltpu.VMEM((TOK_PER_W, D), jnp.float32),
            pltpu.SemaphoreType.DMA,
            pltpu.SemaphoreType.DMA,
        ],
    )
    def k(h_hbm, d0_hbm, d1_hbm, x_hbm, i0, i1, buf, s0, s1):
        wid = lax.axis_index("s") * 2 + lax.axis_index("c")
        base = wid * TOK_PER_W
        pltpu.sync_copy(d0_hbm.at[wid], i0)
        pltpu.sync_copy(d1_hbm.at[wid], i1)
        pltpu.sync_copy(h_hbm.at[pl.ds(base, TOK_PER_W)], buf)
        da = pltpu.async_copy(buf, x_hbm.at[i0], s0)
        db = pltpu.async_copy(buf, x_hbm.at[i1], s1)
        da.wait()
        db.wait()

    return k(h2, dest0_2, dest1_2)


def _tc_expert_mlp(x, w1, b1, w2, b2, tile_e, tile_active, nt_dyn):
    def body(te_ref, ta_ref, x_ref, w1_ref, b1_ref, w2_ref, b2_ref, y_ref):
        t = pl.program_id(0)
        f = pl.program_id(1)

        @pl.when(f == 0)
        def _():
            y_ref[...] = jnp.broadcast_to(b2_ref[0, 0][None, :], (T, D))

        @pl.when(ta_ref[t] == 1)
        def _():
            h = jnp.dot(x_ref[...].astype(jnp.bfloat16),
                        w1_ref[0].astype(jnp.bfloat16),
                        preferred_element_type=jnp.float32)
            h = jnp.maximum(h + b1_ref[0, 0][None, :], 0.0)
            y_ref[...] += jnp.dot(h.astype(jnp.bfloat16),
                                  w2_ref[0].astype(jnp.bfloat16),
                                  preferred_element_type=jnp.float32)

    grid_spec = pltpu.PrefetchScalarGridSpec(
        num_scalar_prefetch=2,
        grid=(nt_dyn, NF),
        in_specs=[
            pl.BlockSpec((T, D), lambda t, f, te, ta: (t, 0)),
            pl.BlockSpec((1, D, FB), lambda t, f, te, ta: (te[t], 0, f)),
            pl.BlockSpec((1, 1, FB), lambda t, f, te, ta: (te[t] * NF + f, 0, 0)),
            pl.BlockSpec((1, FB, D), lambda t, f, te, ta: (te[t], f, 0)),
            pl.BlockSpec((1, 1, D), lambda t, f, te, ta: (te[t], 0, 0)),
        ],
        out_specs=pl.BlockSpec((T, D), lambda t, f, te, ta: (t, 0)),
    )
    return pl.pallas_call(
        body,
        grid_spec=grid_spec,
        out_shape=jax.ShapeDtypeStruct((R, D), jnp.float32),
        compiler_params=pltpu.CompilerParams(
            dimension_semantics=("arbitrary", "arbitrary")),
    )(tile_e, tile_active, x, w1, b1.reshape(E * NF, 1, FB), w2,
      b2.reshape(E, 1, D))


def _sc_combine(y, pos0_2, pos1_2, w0b, w1b):
    NCC = TOK_PER_W // CCH

    @functools.partial(
        pl.kernel, mesh=_mesh(),
        out_type=jax.ShapeDtypeStruct((S, D), jnp.float32),
        scratch_types=[
            pltpu.VMEM((TOK_PER_W,), jnp.int32),
            pltpu.VMEM((TOK_PER_W,), jnp.int32),
            pltpu.VMEM((TOK_PER_W, 16), jnp.float32),
            pltpu.VMEM((TOK_PER_W, 16), jnp.float32),
        ] + [pltpu.VMEM((CCH, D), jnp.float32) for _ in range(6)]
          + [pltpu.SemaphoreType.DMA for _ in range(9)],
    )
    def k(y_hbm, p0_hbm, p1_hbm, w0_hbm, w1_hbm, out_hbm, p0v, p1v, w0v, w1v, *sc):
        abuf, bbuf = sc[0:3], sc[3:6]
        asem, bsem, osem = sc[6:9], sc[9:12], sc[12:15]
        wid = lax.axis_index("s") * 2 + lax.axis_index("c")
        base = wid * TOK_PER_W
        pltpu.sync_copy(p0_hbm.at[wid], p0v)
        pltpu.sync_copy(p1_hbm.at[wid], p1v)
        pltpu.sync_copy(w0_hbm.at[wid], w0v)
        pltpu.sync_copy(w1_hbm.at[wid], w1v)

        def start(c):
            p = c % 3
            sl = pl.ds(c * CCH, CCH)
            da = pltpu.async_copy(y_hbm.at[p0v.at[sl]], abuf[p], asem[p])
            db = pltpu.async_copy(y_hbm.at[p1v.at[sl]], bbuf[p], bsem[p])
            return da, db

        descs = {0: start(0), 1: start(1)}
        stores = {}
        for c in range(NCC):
            if c + 2 < NCC:
                if c - 1 in stores:
                    stores.pop(c - 1).wait()
                descs[c + 2] = start(c + 2)
            descs[c][0].wait()
            descs[c][1].wait()
            p = c % 3
            av, bv = abuf[p], bbuf[p]

            def row(i, carry, av=av, bv=bv, c=c):
                w0r = w0v[c * CCH + i, :]
                w1r = w1v[c * CCH + i, :]
                for j in range(D // 16):
                    sl = pl.ds(j * 16, 16)
                    av[i, sl] = av[i, sl] * w0r + bv[i, sl] * w1r
                return carry

            lax.fori_loop(0, CCH, row, 0)
            stores[c] = pltpu.async_copy(
                av, out_hbm.at[pl.ds(base + c * CCH, CCH)], osem[p])
        for d in stores.values():
            d.wait()

    return k(y, pos0_2, pos1_2, w0b, w1b)


def kernel(hidden, cat_indices, sub_indices, weights, W1, b1, W2, b2):
    h2 = hidden.reshape(S, D)
    tile_e, tile_active, dest, total = _routing(cat_indices, sub_indices, weights)

    pos = dest.reshape(S, K)
    dest0_2 = pos[:, 0].reshape(NW, TOK_PER_W)
    dest1_2 = pos[:, 1].reshape(NW, TOK_PER_W)
    x = _sc_scatter_x(h2, dest0_2, dest1_2)

    y = _tc_expert_mlp(x, W1, b1, W2, b2, tile_e, tile_active, total // T)

    wflat = weights.reshape(S, K)
    w0b = jnp.broadcast_to(wflat[:, 0][:, None], (S, 16)).reshape(
        NW, TOK_PER_W, 16)
    w1b = jnp.broadcast_to(wflat[:, 1][:, None], (S, 16)).reshape(
        NW, TOK_PER_W, 16)
    out = _sc_combine(y, dest0_2, dest1_2, w0b, w1b)
    return out.reshape(hidden.shape)
